# baseline (speedup 1.0000x reference)
import jax
import jax.numpy as jnp
from jax.experimental import pallas as pl
from jax.experimental.pallas import tpu as pltpu

_N = 50000
_M = 16
_H = 128
_BLK = 5000
_NB = _N // _BLK


def _fused(e_ref, m_ref, w_ref, b_ref, out_ref, s_acc, cnt_acc, he_s):
    i = pl.program_id(0)

    @pl.when(i == 0)
    def _init():
        s_acc[...] = jnp.zeros_like(s_acc)
        cnt_acc[...] = jnp.zeros_like(cnt_acc)

    @pl.when(i < _NB)
    def _reduce():
        maskf = m_ref[pl.ds(i * _BLK, _BLK), :]
        s_acc[...] += jax.lax.dot_general(
            maskf, e_ref[...], (((0,), (0,)), ((), ())),
            preferred_element_type=jnp.float32)
        cnt_acc[...] += jnp.sum(maskf, axis=0)[None, :]

    @pl.when(i == _NB - 1)
    def _epilogue():
        cnt = cnt_acc[0, :]
        binv = jnp.where(cnt > 0, 1.0 / cnt, 0.0)
        sw = jax.lax.dot_general(
            s_acc[...], w_ref[...], (((1,), (1,)), ((), ())),
            preferred_element_type=jnp.float32)
        he_s[...] = binv[:, None] * sw

    @pl.when(i >= _NB)
    def _emit():
        maskf = m_ref[pl.ds((i - _NB) * _BLK, _BLK), :]
        d = jnp.sum(maskf, axis=1)
        dinv = jnp.where(d > 0, 1.0 / d, 0.0)
        y = jnp.dot(maskf, he_s[...], preferred_element_type=jnp.float32)
        y = y * dinv[:, None] + b_ref[...]
        out_ref[...] = jnp.where(y >= 0, y, 0.01 * y)


def kernel(e_s, industry_matrix, W, b):
    maskf = industry_matrix.astype(jnp.float32)
    out = pl.pallas_call(
        _fused,
        grid=(2 * _NB,),
        in_specs=[
            pl.BlockSpec((_BLK, _H), lambda i: (jnp.minimum(i, _NB - 1), 0)),
            pl.BlockSpec((_N, _M), lambda i: (0, 0)),
            pl.BlockSpec((_H, _H), lambda i: (0, 0)),
            pl.BlockSpec((1, _H), lambda i: (0, 0)),
        ],
        out_specs=pl.BlockSpec(
            (_BLK, _H), lambda i: (jnp.maximum(i - _NB, 0), 0)),
        out_shape=jax.ShapeDtypeStruct((_N, _H), jnp.float32),
        scratch_shapes=[
            pltpu.VMEM((_M, _H), jnp.float32),
            pltpu.VMEM((1, _M), jnp.float32),
            pltpu.VMEM((_M, _H), jnp.float32),
        ],
    )(e_s, maskf, W, b.reshape(1, _H))
    return out


# speedup vs baseline: 1.0846x; 1.0846x over previous
import jax
import jax.numpy as jnp
from jax.experimental import pallas as pl
from jax.experimental.pallas import tpu as pltpu

_N = 50000
_M = 16
_H = 128
_BLK = 10000
_NB = _N // _BLK


def _fused(e_ref, m_ref, w_ref, b_ref, out_ref, s_acc, cnt_acc, he_s):
    i = pl.program_id(0)

    @pl.when(i == 0)
    def _init():
        s_acc[...] = jnp.zeros_like(s_acc)
        cnt_acc[...] = jnp.zeros_like(cnt_acc)

    @pl.when(i < _NB)
    def _reduce():
        maskf = m_ref[pl.ds(i * _BLK, _BLK), :]
        s_acc[...] += jax.lax.dot_general(
            maskf, e_ref[...], (((0,), (0,)), ((), ())),
            preferred_element_type=jnp.float32)
        cnt_acc[...] += jnp.sum(maskf, axis=0)[None, :]

    @pl.when(i == _NB - 1)
    def _epilogue():
        cnt = cnt_acc[0, :]
        binv = jnp.where(cnt > 0, 1.0 / cnt, 0.0)
        sw = jax.lax.dot_general(
            s_acc[...], w_ref[...], (((1,), (1,)), ((), ())),
            preferred_element_type=jnp.float32)
        he_s[...] = binv[:, None] * sw

    @pl.when(i >= _NB)
    def _emit():
        maskf = m_ref[pl.ds((i - _NB) * _BLK, _BLK), :]
        d = jnp.sum(maskf, axis=1)
        dinv = jnp.where(d > 0, 1.0 / d, 0.0)
        y = jnp.dot(maskf, he_s[...], preferred_element_type=jnp.float32)
        y = y * dinv[:, None] + b_ref[...]
        out_ref[...] = jnp.where(y >= 0, y, 0.01 * y)


def kernel(e_s, industry_matrix, W, b):
    maskf = industry_matrix.astype(jnp.float32)
    out = pl.pallas_call(
        _fused,
        grid=(2 * _NB,),
        in_specs=[
            pl.BlockSpec((_BLK, _H), lambda i: (jnp.minimum(i, _NB - 1), 0)),
            pl.BlockSpec((_N, _M), lambda i: (0, 0)),
            pl.BlockSpec((_H, _H), lambda i: (0, 0)),
            pl.BlockSpec((1, _H), lambda i: (0, 0)),
        ],
        out_specs=pl.BlockSpec(
            (_BLK, _H), lambda i: (jnp.maximum(i - _NB, 0), 0)),
        out_shape=jax.ShapeDtypeStruct((_N, _H), jnp.float32),
        scratch_shapes=[
            pltpu.VMEM((_M, _H), jnp.float32),
            pltpu.VMEM((1, _M), jnp.float32),
            pltpu.VMEM((_M, _H), jnp.float32),
        ],
    )(e_s, maskf, W, b.reshape(1, _H))
    return out


# speedup vs baseline: 1.0966x; 1.0111x over previous
import jax
import jax.numpy as jnp
from jax.experimental import pallas as pl
from jax.experimental.pallas import tpu as pltpu

_N = 50000
_M = 16
_H = 128
_BLK = 10000
_NB = _N // _BLK


def _fused(e_ref, m_ref, w_ref, b_ref, out_ref, s_acc, g_acc, he_s):
    i = pl.program_id(0)

    @pl.when(i == 0)
    def _init():
        s_acc[...] = jnp.zeros_like(s_acc)
        g_acc[...] = jnp.zeros_like(g_acc)

    @pl.when(i < _NB)
    def _reduce():
        maskf = m_ref[pl.ds(i * _BLK, _BLK), :].astype(jnp.float32)
        s_acc[...] += jax.lax.dot_general(
            maskf, e_ref[...], (((0,), (0,)), ((), ())),
            preferred_element_type=jnp.float32)
        g_acc[...] += jax.lax.dot_general(
            maskf, maskf, (((0,), (0,)), ((), ())),
            preferred_element_type=jnp.float32)

    @pl.when(i == _NB - 1)
    def _epilogue():
        row = jax.lax.broadcasted_iota(jnp.int32, (_M, _M), 0)
        col = jax.lax.broadcasted_iota(jnp.int32, (_M, _M), 1)
        gdiag = jnp.where(row == col, g_acc[...], 0.0)
        bb = jnp.dot(gdiag, jnp.ones((_M, _H), jnp.float32),
                     preferred_element_type=jnp.float32)
        sw = jax.lax.dot_general(
            s_acc[...], w_ref[...], (((1,), (1,)), ((), ())),
            preferred_element_type=jnp.float32)
        he_s[...] = jnp.where(bb > 0, sw / bb, 0.0)

    @pl.when(i >= _NB)
    def _emit():
        maskf = m_ref[pl.ds((i - _NB) * _BLK, _BLK), :].astype(jnp.float32)
        dbc = jnp.dot(maskf, jnp.ones((_M, _H), jnp.float32),
                      preferred_element_type=jnp.float32)
        y = jnp.dot(maskf, he_s[...], preferred_element_type=jnp.float32)
        y = jnp.where(dbc > 0, y / dbc, 0.0) + b_ref[...]
        out_ref[...] = jnp.where(y >= 0, y, 0.01 * y)


def kernel(e_s, industry_matrix, W, b):
    out = pl.pallas_call(
        _fused,
        grid=(2 * _NB,),
        in_specs=[
            pl.BlockSpec((_BLK, _H), lambda i: (jnp.minimum(i, _NB - 1), 0)),
            pl.BlockSpec((_N, _M), lambda i: (0, 0)),
            pl.BlockSpec((_H, _H), lambda i: (0, 0)),
            pl.BlockSpec((1, _H), lambda i: (0, 0)),
        ],
        out_specs=pl.BlockSpec(
            (_BLK, _H), lambda i: (jnp.maximum(i - _NB, 0), 0)),
        out_shape=jax.ShapeDtypeStruct((_N, _H), jnp.float32),
        scratch_shapes=[
            pltpu.VMEM((_M, _H), jnp.float32),
            pltpu.VMEM((_M, _M), jnp.float32),
            pltpu.VMEM((_M, _H), jnp.float32),
        ],
    )(e_s, industry_matrix, W, b.reshape(1, _H))
    return out


# speedup vs baseline: 1.0984x; 1.0016x over previous
import jax
import jax.numpy as jnp
from jax.experimental import pallas as pl
from jax.experimental.pallas import tpu as pltpu

_N = 50000
_M = 16
_H = 128
_BLK = 10000
_NB = _N // _BLK


def _fused(e_ref, m_ref, w_ref, b_ref, out_ref, s_acc, g_acc, he_s):
    i = pl.program_id(0)

    @pl.when(i == 0)
    def _init():
        s_acc[...] = jnp.zeros_like(s_acc)
        g_acc[...] = jnp.zeros_like(g_acc)

    @pl.when(i < _NB)
    def _reduce():
        maskf = m_ref[pl.ds(i * _BLK, _BLK), :]
        s_acc[...] += jax.lax.dot_general(
            maskf, e_ref[...], (((0,), (0,)), ((), ())),
            preferred_element_type=jnp.float32)
        g_acc[...] += jax.lax.dot_general(
            maskf, maskf, (((0,), (0,)), ((), ())),
            preferred_element_type=jnp.float32)

    @pl.when(i == _NB - 1)
    def _epilogue():
        row = jax.lax.broadcasted_iota(jnp.int32, (_M, _M), 0)
        col = jax.lax.broadcasted_iota(jnp.int32, (_M, _M), 1)
        gdiag = jnp.where(row == col, g_acc[...], 0.0)
        bb = jnp.dot(gdiag, jnp.ones((_M, _H), jnp.float32),
                     preferred_element_type=jnp.float32)
        sw = jax.lax.dot_general(
            s_acc[...], w_ref[...], (((1,), (1,)), ((), ())),
            preferred_element_type=jnp.float32)
        he_s[...] = jnp.where(bb > 0, sw / bb, 0.0)

    @pl.when(i >= _NB)
    def _emit():
        maskf = m_ref[pl.ds((i - _NB) * _BLK, _BLK), :]
        dbc = jnp.dot(maskf, jnp.ones((_M, _H), jnp.float32),
                      preferred_element_type=jnp.float32)
        y = jnp.dot(maskf, he_s[...], preferred_element_type=jnp.float32)
        y = jnp.where(dbc > 0, y / dbc, 0.0) + b_ref[...]
        out_ref[...] = jnp.where(y >= 0, y, 0.01 * y)


def kernel(e_s, industry_matrix, W, b):
    maskf = industry_matrix.astype(jnp.float32)
    out = pl.pallas_call(
        _fused,
        grid=(2 * _NB,),
        in_specs=[
            pl.BlockSpec((_BLK, _H), lambda i: (jnp.minimum(i, _NB - 1), 0)),
            pl.BlockSpec((_N, _M), lambda i: (0, 0)),
            pl.BlockSpec((_H, _H), lambda i: (0, 0)),
            pl.BlockSpec((1, _H), lambda i: (0, 0)),
        ],
        out_specs=pl.BlockSpec(
            (_BLK, _H), lambda i: (jnp.maximum(i - _NB, 0), 0)),
        out_shape=jax.ShapeDtypeStruct((_N, _H), jnp.float32),
        scratch_shapes=[
            pltpu.VMEM((_M, _H), jnp.float32),
            pltpu.VMEM((_M, _M), jnp.float32),
            pltpu.VMEM((_M, _H), jnp.float32),
        ],
    )(e_s, maskf, W, b.reshape(1, _H))
    return out
